# baseline (speedup 1.0000x reference)
import jax
import jax.numpy as jnp
from jax import lax
from jax.experimental import pallas as pl
from jax.experimental.pallas import tpu as pltpu

_B = 128
_K = 1024
_DIM = 50257
_DC = 4096
_NS = (_DIM + _DC - 1) // _DC


def _body(q_ref, lk_ref, lab_ref, out_ref, cross_acc, selfvec_acc):
    i = pl.program_id(0)

    @pl.when(i == 0)
    def _init():
        cross_acc[...] = jnp.zeros_like(cross_acc)
        selfvec_acc[...] = jnp.zeros_like(selfvec_acc)

    d_base = i * _DC
    col = lax.broadcasted_iota(jnp.int32, (1, _DC), 1) + d_base
    valid = col < _DIM
    lkv = jnp.where(valid, lk_ref[...], -1e30)
    e = jnp.exp(lkv)
    logq = jnp.where(valid, jnp.log(q_ref[...] + 1e-10), 0.0)

    dn = (((1,), (1,)), ((), ()))
    cross_acc[...] += lax.dot_general(
        logq.astype(jnp.bfloat16), e.astype(jnp.bfloat16), dn,
        preferred_element_type=jnp.float32)

    elk = e * lkv
    acc = selfvec_acc[...]
    for t in range(_DC // 128):
        acc = acc + elk[:, t * 128:(t + 1) * 128]
    selfvec_acc[...] = acc

    @pl.when(i == _NS - 1)
    def _finish():
        self_col = jnp.sum(selfvec_acc[...], axis=1, keepdims=True)
        self_row = jnp.transpose(self_col)
        scores = cross_acc[...] - self_row
        iota_k = lax.broadcasted_iota(jnp.int32, (_B, _K), 1)
        labels = lab_ref[...]
        total = jnp.zeros((_B, 1), jnp.int32)
        s = scores
        for _ in range(3):
            m = jnp.max(s, axis=1, keepdims=True)
            first = jnp.min(jnp.where(s == m, iota_k, _K), axis=1, keepdims=True)
            sel = iota_k == first
            total += jnp.sum(jnp.where(sel, labels, 0), axis=1, keepdims=True)
            s = jnp.where(sel, -jnp.inf, s)
        out_ref[...] = (total >= 2).astype(jnp.int32)


_knn_vote = pl.pallas_call(
    _body,
    grid=(_NS,),
    in_specs=[
        pl.BlockSpec((_B, _DC), lambda i: (0, i)),
        pl.BlockSpec((_K, _DC), lambda i: (0, i)),
        pl.BlockSpec((1, _K), lambda i: (0, 0)),
    ],
    out_specs=pl.BlockSpec((_B, 1), lambda i: (0, 0)),
    out_shape=jax.ShapeDtypeStruct((_B, 1), jnp.int32),
    scratch_shapes=[
        pltpu.VMEM((_B, _K), jnp.float32),
        pltpu.VMEM((_K, 128), jnp.float32),
    ],
    compiler_params=pltpu.CompilerParams(dimension_semantics=("arbitrary",)),
)


def kernel(query, log_k, labels):
    labels2 = labels.astype(jnp.int32).reshape(1, _K)
    out = _knn_vote(query, log_k, labels2)
    return out.reshape(_B)


# speedup vs baseline: 1.0035x; 1.0035x over previous
import jax
import jax.numpy as jnp
from jax import lax
from jax.experimental import pallas as pl
from jax.experimental.pallas import tpu as pltpu

_B = 128
_K = 1024
_DIM = 50257
_DC = 4096
_NS = (_DIM + _DC - 1) // _DC


def _body(q_ref, lk_ref, lab_ref, out_ref, cross_acc, selfvec_acc):
    i = pl.program_id(0)

    @pl.when(i == 0)
    def _init():
        cross_acc[...] = jnp.zeros_like(cross_acc)
        selfvec_acc[...] = jnp.zeros_like(selfvec_acc)

    dn = (((1,), (1,)), ((), ()))

    def _accumulate(lkv, logq):
        e = jnp.exp(lkv)
        cross_acc[...] += lax.dot_general(
            logq.astype(jnp.bfloat16), e.astype(jnp.bfloat16), dn,
            preferred_element_type=jnp.float32)
        elk = e * lkv
        acc = selfvec_acc[...]
        for t in range(_DC // 128):
            acc = acc + elk[:, t * 128:(t + 1) * 128]
        selfvec_acc[...] = acc

    @pl.when(i < _NS - 1)
    def _full():
        _accumulate(lk_ref[...], jnp.log(q_ref[...] + 1e-10))

    @pl.when(i == _NS - 1)
    def _last():
        col = lax.broadcasted_iota(jnp.int32, (1, _DC), 1) + i * _DC
        valid = col < _DIM
        lkv = jnp.where(valid, lk_ref[...], -1e30)
        logq = jnp.where(valid, jnp.log(q_ref[...] + 1e-10), 0.0)
        _accumulate(lkv, logq)

    @pl.when(i == _NS - 1)
    def _finish():
        self_col = jnp.sum(selfvec_acc[...], axis=1, keepdims=True)
        self_row = jnp.transpose(self_col)
        scores = cross_acc[...] - self_row
        iota_k = lax.broadcasted_iota(jnp.int32, (_B, _K), 1)
        labels = lab_ref[...]
        total = jnp.zeros((_B, 1), jnp.int32)
        s = scores
        for _ in range(3):
            m = jnp.max(s, axis=1, keepdims=True)
            first = jnp.min(jnp.where(s == m, iota_k, _K), axis=1, keepdims=True)
            sel = iota_k == first
            total += jnp.sum(jnp.where(sel, labels, 0), axis=1, keepdims=True)
            s = jnp.where(sel, -jnp.inf, s)
        out_ref[...] = (total >= 2).astype(jnp.int32)


_knn_vote = pl.pallas_call(
    _body,
    grid=(_NS,),
    in_specs=[
        pl.BlockSpec((_B, _DC), lambda i: (0, i)),
        pl.BlockSpec((_K, _DC), lambda i: (0, i)),
        pl.BlockSpec((1, _K), lambda i: (0, 0)),
    ],
    out_specs=pl.BlockSpec((_B, 1), lambda i: (0, 0)),
    out_shape=jax.ShapeDtypeStruct((_B, 1), jnp.int32),
    scratch_shapes=[
        pltpu.VMEM((_B, _K), jnp.float32),
        pltpu.VMEM((_K, 128), jnp.float32),
    ],
    compiler_params=pltpu.CompilerParams(dimension_semantics=("arbitrary",)),
)


def kernel(query, log_k, labels):
    labels2 = labels.astype(jnp.int32).reshape(1, _K)
    out = _knn_vote(query, log_k, labels2)
    return out.reshape(_B)


# speedup vs baseline: 1.0635x; 1.0599x over previous
import jax
import jax.numpy as jnp
from jax import lax
from jax.experimental import pallas as pl
from jax.experimental.pallas import tpu as pltpu

_B = 128
_K = 1024
_DIM = 50257
_KT = 64
_NT = _K // _KT
_DC = 4096
_NS = (_DIM + _DC - 1) // _DC


def _logq_body(q_ref, out_ref):
    out_ref[...] = jnp.log(q_ref[...] + 1e-10).astype(jnp.bfloat16)


_logq = pl.pallas_call(
    _logq_body,
    out_shape=jax.ShapeDtypeStruct((_B, _DIM), jnp.bfloat16),
    compiler_params=pltpu.CompilerParams(vmem_limit_bytes=50 * 1024 * 1024),
)


def _main_body(lq_ref, lk_ref, lab_ref, out_ref, scoresT, selfvec):
    i = pl.program_id(0)
    dn = (((1,), (1,)), ((), ()))
    acc = jnp.zeros((_KT, _B), jnp.float32)
    sacc = jnp.zeros((_KT, 128), jnp.float32)
    for c in range(_NS):
        lo = c * _DC
        hi = min(lo + _DC, _DIM)
        lkv = lk_ref[0, :, lo:hi]
        if hi - lo < _DC:
            lkv = lax.pad(lkv, jnp.float32(-1e30), ((0, 0, 0), (0, _DC - (hi - lo), 0)))
        e = jnp.exp(lkv)
        acc = acc + lax.dot_general(
            e.astype(jnp.bfloat16), lq_ref[:, lo:lo + _DC] if hi - lo == _DC
            else lax.pad(lq_ref[:, lo:hi], jnp.bfloat16(0.0),
                         ((0, 0, 0), (0, _DC - (hi - lo), 0))),
            dn, preferred_element_type=jnp.float32)
        elk = e * lkv
        for t in range(_DC // 128):
            sacc = sacc + elk[:, t * 128:(t + 1) * 128]
    self_col = jnp.sum(sacc, axis=1, keepdims=True)
    scoresT[pl.ds(i * _KT, _KT), :] = acc - self_col

    @pl.when(i == _NT - 1)
    def _finish():
        s = scoresT[...]
        iota_k = lax.broadcasted_iota(jnp.int32, (_K, _B), 0)
        labels = lab_ref[...]
        total = jnp.zeros((1, _B), jnp.int32)
        for _ in range(3):
            m = jnp.max(s, axis=0, keepdims=True)
            first = jnp.min(jnp.where(s == m, iota_k, _K), axis=0, keepdims=True)
            sel = iota_k == first
            total += jnp.sum(jnp.where(sel, labels, 0), axis=0, keepdims=True)
            s = jnp.where(sel, -jnp.inf, s)
        out_ref[...] = (total >= 2).astype(jnp.int32)


_knn_vote = pl.pallas_call(
    _main_body,
    grid=(_NT,),
    in_specs=[
        pl.BlockSpec((_B, _DIM), lambda i: (0, 0)),
        pl.BlockSpec((1, _KT, _DIM), lambda i: (i, 0, 0)),
        pl.BlockSpec((_K, 1), lambda i: (0, 0)),
    ],
    out_specs=pl.BlockSpec((1, _B), lambda i: (0, 0)),
    out_shape=jax.ShapeDtypeStruct((1, _B), jnp.int32),
    scratch_shapes=[
        pltpu.VMEM((_K, _B), jnp.float32),
        pltpu.VMEM((_KT, 128), jnp.float32),
    ],
    compiler_params=pltpu.CompilerParams(
        dimension_semantics=("arbitrary",), vmem_limit_bytes=60 * 1024 * 1024),
)


def kernel(query, log_k, labels):
    lq = _logq(query)
    labels2 = labels.astype(jnp.int32).reshape(_K, 1)
    out = _knn_vote(lq, log_k.reshape(_NT, _KT, _DIM), labels2)
    return out.reshape(_B)
